# Initial kernel scaffold; baseline (speedup 1.0000x reference)
import jax
import jax.numpy as jnp
from jax.experimental import pallas as pl


def kernel(x, gate_weight):
    raise NotImplementedError("write your pallas kernel here")


# speedup vs baseline: 2.1743x; 2.1743x over previous
import functools

import jax
import jax.numpy as jnp
from jax import lax
from jax.experimental import pallas as pl
from jax.experimental.pallas import tpu as pltpu
from jax.experimental.pallas import tpu_sc as plsc

_H = 768
_E = 64
_N = 32768
_NW = 32
_TPW = _N // _NW
_L = 16


def _mm_body(w_ref, x_ref, o_ref):
    o_ref[0] = lax.dot_general(
        w_ref[...], x_ref[...],
        dimension_numbers=(((1,), (1,)), ((), ())),
        preferred_element_type=jnp.float32,
    )


def _matmul_logits_t(x, gate_weight):
    return pl.pallas_call(
        _mm_body,
        grid=(_NW,),
        in_specs=[
            pl.BlockSpec((_E, _H), lambda i: (0, 0)),
            pl.BlockSpec((_TPW, _H), lambda i: (i, 0)),
        ],
        out_specs=pl.BlockSpec((1, _E, _TPW), lambda i: (i, 0, 0)),
        out_shape=jax.ShapeDtypeStruct((_NW, _E, _TPW), jnp.float32),
    )(gate_weight, x)


def _topk_sc_body(l_hbm, g1_hbm, g2_hbm, i1_hbm, i2_hbm,
                  blk, g1v, g2v, i1v, i2v):
    wid = lax.axis_index("s") * 2 + lax.axis_index("c")
    pltpu.sync_copy(l_hbm.at[wid], blk)

    def group(g, _):
        t0 = g * _L
        m1 = blk[0, pl.ds(t0, _L)]
        i1 = jnp.zeros((_L,), jnp.int32)
        m2 = jnp.full((_L,), -jnp.inf, jnp.float32)
        i2 = jnp.zeros((_L,), jnp.int32)
        for e in range(1, _E):
            v = blk[e, pl.ds(t0, _L)]
            ev = jnp.full((_L,), e, jnp.int32)
            gt1 = v > m1
            gt2 = v > m2
            m2 = jnp.where(gt1, m1, jnp.where(gt2, v, m2))
            i2 = jnp.where(gt1, i1, jnp.where(gt2, ev, i2))
            m1 = jnp.where(gt1, v, m1)
            i1 = jnp.where(gt1, ev, i1)
        ed = jnp.exp(m2 - m1)
        g1 = 1.0 / (1.0 + ed)
        g1v[pl.ds(t0, _L)] = g1
        g2v[pl.ds(t0, _L)] = 1.0 - g1
        i1v[pl.ds(t0, _L)] = i1
        i2v[pl.ds(t0, _L)] = i2
        return ()

    lax.fori_loop(0, _TPW // _L, group, ())

    base = wid * _TPW
    pltpu.sync_copy(g1v, g1_hbm.at[pl.ds(base, _TPW)])
    pltpu.sync_copy(g2v, g2_hbm.at[pl.ds(base, _TPW)])
    pltpu.sync_copy(i1v, i1_hbm.at[pl.ds(base, _TPW)])
    pltpu.sync_copy(i2v, i2_hbm.at[pl.ds(base, _TPW)])


def _topk_sc(logits_t):
    mesh = plsc.VectorSubcoreMesh(core_axis_name="c", subcore_axis_name="s")
    f = functools.partial(
        pl.kernel,
        mesh=mesh,
        out_type=[
            jax.ShapeDtypeStruct((_N,), jnp.float32),
            jax.ShapeDtypeStruct((_N,), jnp.float32),
            jax.ShapeDtypeStruct((_N,), jnp.int32),
            jax.ShapeDtypeStruct((_N,), jnp.int32),
        ],
        scratch_types=[
            pltpu.VMEM((_E, _TPW), jnp.float32),
            pltpu.VMEM((_TPW,), jnp.float32),
            pltpu.VMEM((_TPW,), jnp.float32),
            pltpu.VMEM((_TPW,), jnp.int32),
            pltpu.VMEM((_TPW,), jnp.int32),
        ],
    )(_topk_sc_body)
    return f(logits_t)


def kernel(x, gate_weight):
    logits_t = _matmul_logits_t(x, gate_weight)
    g1, g2, i1, i2 = _topk_sc(logits_t)
    gates = jnp.stack([g1, g2], axis=-1)
    idx = jnp.stack([i1, i2], axis=-1)
    return (gates, idx)
